# Initial kernel scaffold; baseline (speedup 1.0000x reference)
import jax
import jax.numpy as jnp
from jax.experimental import pallas as pl


def kernel(state, edge_index, edge_attr, W, b, gamma, beta, lin_W, lin_b):
    raise NotImplementedError("write your pallas kernel here")


# speedup vs baseline: 97.2635x; 97.2635x over previous
import functools

import jax
import jax.numpy as jnp
from jax import lax
from jax.experimental import pallas as pl
from jax.experimental.pallas import tpu as pltpu
from jax.experimental.pallas import tpu_sc as plsc

_LANES = 16
_CH = 3200
_NWORKERS = 32


def _round_up(x, m):
    return (x + m - 1) // m * m


def _zero_ref(ref, n):
    zeros = jnp.zeros((_LANES,), jnp.float32)

    def body(i, c):
        ref[pl.ds(i * _LANES, _LANES)] = zeros
        return c

    lax.fori_loop(0, n // _LANES, body, None)


def _make_deg_kernel(np_, ep):
    epw = ep // _NWORKERS
    nch = epw // _CH
    mesh = plsc.VectorSubcoreMesh(core_axis_name="c", subcore_axis_name="s")

    @functools.partial(
        pl.kernel,
        out_type=jax.ShapeDtypeStruct((_NWORKERS, np_), jnp.float32),
        mesh=mesh,
        scratch_types=[
            pltpu.VMEM((np_,), jnp.float32),
            pltpu.VMEM((_CH,), jnp.int32),
        ],
        compiler_params=pltpu.CompilerParams(needs_layout_passes=False),
    )
    def deg_kernel(dst_hbm, out_hbm, acc, dstbuf):
        wid = lax.axis_index("c") * 16 + lax.axis_index("s")
        _zero_ref(acc, np_)
        ones = jnp.ones((_LANES,), jnp.float32)
        base = wid * epw

        def chunk(ci, cry):
            pltpu.sync_copy(dst_hbm.at[pl.ds(base + ci * _CH, _CH)], dstbuf)

            def inner(j, c2):
                idx = dstbuf[pl.ds(j * _LANES, _LANES)]
                plsc.addupdate_scatter(acc, [idx], ones)
                return c2

            lax.fori_loop(0, _CH // _LANES, inner, None)
            return cry

        lax.fori_loop(0, nch, chunk, None)
        pltpu.sync_copy(acc, out_hbm.at[wid])

    return deg_kernel


def _make_agg_kernel(np_, ep):
    epw = ep // _NWORKERS
    nch = epw // _CH
    mesh = plsc.VectorSubcoreMesh(core_axis_name="c", subcore_axis_name="s")

    @functools.partial(
        pl.kernel,
        out_type=jax.ShapeDtypeStruct((_NWORKERS, np_), jnp.float32),
        mesh=mesh,
        scratch_types=[
            pltpu.VMEM((np_,), jnp.float32),
            pltpu.VMEM((np_,), jnp.float32),
            pltpu.VMEM((_CH,), jnp.int32),
            pltpu.VMEM((_CH,), jnp.int32),
        ],
        compiler_params=pltpu.CompilerParams(needs_layout_passes=False),
    )
    def agg_kernel(src_hbm, dst_hbm, u_hbm, out_hbm, u, tacc, srcbuf, dstbuf):
        wid = lax.axis_index("c") * 16 + lax.axis_index("s")
        _zero_ref(tacc, np_)
        pltpu.sync_copy(u_hbm, u)
        base = wid * epw

        def chunk(ci, cry):
            eb = base + ci * _CH
            pltpu.sync_copy(src_hbm.at[pl.ds(eb, _CH)], srcbuf)
            pltpu.sync_copy(dst_hbm.at[pl.ds(eb, _CH)], dstbuf)

            def inner(j, c2):
                ds = pl.ds(j * _LANES, _LANES)
                vals = plsc.load_gather(u, [srcbuf[ds]])
                plsc.addupdate_scatter(tacc, [dstbuf[ds]], vals)
                return c2

            lax.fori_loop(0, _CH // _LANES, inner, None)
            return cry

        lax.fori_loop(0, nch, chunk, None)
        pltpu.sync_copy(tacc, out_hbm.at[wid])

    return agg_kernel


def _u_body(degp_ref, state_ref, u_ref, dis_ref):
    deg = jnp.sum(degp_ref[:], axis=0) + 1.0
    dis = lax.rsqrt(deg)
    dis_ref[:] = dis
    u_ref[:] = dis * state_ref[:]


def _head_body(n, tp_ref, dis_ref, st_ref, w_ref, g_ref, be_ref, lwt_ref,
               lb_ref, a0_ref, a1_ref):
    t = jnp.sum(tp_ref[:], axis=0)
    dis = dis_ref[:]
    s = dis * (t + dis * st_ref[:])
    inv_n = jnp.float32(1.0 / n)
    m = jnp.sum(s) * inv_n
    var = jnp.sum(s * s) * inv_n - m * m
    w = w_ref[:]
    invstd = lax.rsqrt(var * w * w + 1e-5)
    cg = w * invstd * g_ref[:]
    a0c = jnp.sum(cg * lwt_ref[0:1, :])
    a1c = jnp.sum(cg * lwt_ref[1:2, :])
    b0c = jnp.sum(be_ref[:] * lwt_ref[0:1, :]) + lb_ref[0, 0]
    b1c = jnp.sum(be_ref[:] * lwt_ref[1:2, :]) + lb_ref[0, 1]
    z = s - m
    r0 = jnp.maximum(z * a0c + b0c, 0.0)
    r1 = jnp.maximum(z * a1c + b1c, 0.0)
    mx = jnp.maximum(r0, r1)
    e0 = jnp.exp(r0 - mx)
    e1 = jnp.exp(r1 - mx)
    tot = e0 + e1
    a0_ref[:] = e0 / tot
    a1_ref[:] = e1 / tot


def kernel(state, edge_index, edge_attr, W, b, gamma, beta, lin_W, lin_b):
    del edge_attr, b
    n = state.shape[0]
    e = edge_index.shape[1]
    hidden = W.shape[1]
    np_ = _round_up(n, 16 * _CH)
    ep = _round_up(e, _NWORKERS * _CH)
    rows = np_ // 128

    fill = jnp.full((ep - e,), np_ - 1, dtype=edge_index.dtype)
    src_p = jnp.concatenate([edge_index[0], fill])
    dst_p = jnp.concatenate([edge_index[1], fill])
    state_p = jnp.concatenate([state, jnp.zeros((np_ - n,), state.dtype)])
    state_2d = state_p.reshape(rows, 128)

    deg_parts = _make_deg_kernel(np_, ep)(dst_p)

    u2d, dis2d = pl.pallas_call(
        _u_body,
        out_shape=[
            jax.ShapeDtypeStruct((rows, 128), jnp.float32),
            jax.ShapeDtypeStruct((rows, 128), jnp.float32),
        ],
    )(deg_parts.reshape(_NWORKERS, rows, 128), state_2d)

    t_parts = _make_agg_kernel(np_, ep)(src_p, dst_p, u2d.reshape(np_))

    a0, a1 = pl.pallas_call(
        functools.partial(_head_body, n),
        out_shape=[
            jax.ShapeDtypeStruct((rows, 128), jnp.float32),
            jax.ShapeDtypeStruct((rows, 128), jnp.float32),
        ],
    )(
        t_parts.reshape(_NWORKERS, rows, 128),
        dis2d,
        state_2d,
        W.reshape(1, hidden),
        gamma.reshape(1, hidden),
        beta.reshape(1, hidden),
        lin_W.T.reshape(2, hidden),
        lin_b.reshape(1, 2),
    )
    return jnp.stack([a0.reshape(-1)[:n], a1.reshape(-1)[:n]], axis=1)


# speedup vs baseline: 165.0425x; 1.6969x over previous
import functools

import jax
import jax.numpy as jnp
from jax import lax
from jax.experimental import pallas as pl
from jax.experimental.pallas import tpu as pltpu
from jax.experimental.pallas import tpu_sc as plsc

_LANES = 16
_CH = 3200
_NWORKERS = 32
_UNROLL = 4


def _round_up(x, m):
    return (x + m - 1) // m * m


def _zero_ref(ref, n):
    zeros = jnp.zeros((_LANES,), jnp.float32)
    groups = n // _LANES
    uz = 8

    def body(i, c):
        for k in range(uz):
            ref[pl.ds((i * uz + k) * _LANES, _LANES)] = zeros
        return c

    lax.fori_loop(0, groups // uz, body, None)
    for g in range(groups - groups % uz, groups):
        ref[pl.ds(g * _LANES, _LANES)] = zeros


def _emit_groups(ngroups, group_fn):
    main = ngroups - ngroups % _UNROLL

    def body(i, c):
        for k in range(_UNROLL):
            group_fn(i * _UNROLL + k)
        return c

    if main:
        lax.fori_loop(0, main // _UNROLL, body, None)
    for g in range(main, ngroups):
        group_fn(g)


def _make_deg_kernel(np_, e):
    epw = e // _NWORKERS
    nch = epw // _CH
    rem = epw % _CH
    rem_groups = rem // _LANES
    rem_tail = rem % _LANES
    mesh = plsc.VectorSubcoreMesh(core_axis_name="c", subcore_axis_name="s")

    @functools.partial(
        pl.kernel,
        out_type=jax.ShapeDtypeStruct((_NWORKERS, np_), jnp.float32),
        mesh=mesh,
        scratch_types=[
            pltpu.VMEM((np_,), jnp.float32),
            pltpu.VMEM((_CH,), jnp.int32),
        ],
        compiler_params=pltpu.CompilerParams(needs_layout_passes=False),
    )
    def deg_kernel(edge_hbm, out_hbm, acc, dstbuf):
        wid = lax.axis_index("c") * 16 + lax.axis_index("s")
        _zero_ref(acc, np_)
        ones = jnp.ones((_LANES,), jnp.float32)
        base = e + wid * epw

        def scat(g):
            idx = dstbuf[pl.ds(g * _LANES, _LANES)]
            plsc.addupdate_scatter(acc, [idx], ones)

        def chunk(ci, cry):
            pltpu.sync_copy(edge_hbm.at[pl.ds(base + ci * _CH, _CH)],
                            dstbuf)
            _emit_groups(_CH // _LANES, scat)
            return cry

        lax.fori_loop(0, nch, chunk, None)
        if rem:
            pltpu.sync_copy(edge_hbm.at[pl.ds(base + nch * _CH, rem)],
                            dstbuf.at[pl.ds(0, rem)])
            _emit_groups(rem_groups, scat)
            if rem_tail:
                m = lax.iota(jnp.int32, 16) < rem_tail
                idx = dstbuf[pl.ds(rem_groups * _LANES, _LANES)]
                plsc.addupdate_scatter(acc, [idx], ones, mask=m)
        pltpu.sync_copy(acc, out_hbm.at[wid])

    return deg_kernel


def _make_agg_kernel(np_, e):
    epw = e // _NWORKERS
    nch = epw // _CH
    rem = epw % _CH
    rem_groups = rem // _LANES
    rem_tail = rem % _LANES
    mesh = plsc.VectorSubcoreMesh(core_axis_name="c", subcore_axis_name="s")

    @functools.partial(
        pl.kernel,
        out_type=jax.ShapeDtypeStruct((_NWORKERS, np_), jnp.float32),
        mesh=mesh,
        scratch_types=[
            pltpu.VMEM((np_,), jnp.float32),
            pltpu.VMEM((np_,), jnp.float32),
            pltpu.VMEM((_CH,), jnp.int32),
            pltpu.VMEM((_CH,), jnp.int32),
        ],
        compiler_params=pltpu.CompilerParams(needs_layout_passes=False),
    )
    def agg_kernel(edge_hbm, u_hbm, out_hbm, u, tacc, srcbuf, dstbuf):
        wid = lax.axis_index("c") * 16 + lax.axis_index("s")
        _zero_ref(tacc, np_)
        pltpu.sync_copy(u_hbm, u)
        base = wid * epw

        def gs(g):
            ds = pl.ds(g * _LANES, _LANES)
            vals = plsc.load_gather(u, [srcbuf[ds]])
            plsc.addupdate_scatter(tacc, [dstbuf[ds]], vals)

        def chunk(ci, cry):
            eb = base + ci * _CH
            pltpu.sync_copy(edge_hbm.at[pl.ds(eb, _CH)], srcbuf)
            pltpu.sync_copy(edge_hbm.at[pl.ds(e + eb, _CH)], dstbuf)
            _emit_groups(_CH // _LANES, gs)
            return cry

        lax.fori_loop(0, nch, chunk, None)
        if rem:
            eb = base + nch * _CH
            pltpu.sync_copy(edge_hbm.at[pl.ds(eb, rem)],
                            srcbuf.at[pl.ds(0, rem)])
            pltpu.sync_copy(edge_hbm.at[pl.ds(e + eb, rem)],
                            dstbuf.at[pl.ds(0, rem)])
            _emit_groups(rem_groups, gs)
            if rem_tail:
                m = lax.iota(jnp.int32, 16) < rem_tail
                ds = pl.ds(rem_groups * _LANES, _LANES)
                vals = plsc.load_gather(u, [srcbuf[ds]], mask=m)
                plsc.addupdate_scatter(tacc, [dstbuf[ds]], vals, mask=m)
        pltpu.sync_copy(tacc, out_hbm.at[wid])

    return agg_kernel


def _u_body(degp_ref, state_ref, u_ref, dis_ref):
    deg = jnp.sum(degp_ref[:], axis=0) + 1.0
    dis = lax.rsqrt(deg)
    dis_ref[:] = dis
    u_ref[:] = dis * state_ref[:]


def _head_body(n, tp_ref, dis_ref, st_ref, w_ref, g_ref, be_ref, lwt_ref,
               lb_ref, a0_ref, a1_ref):
    t = jnp.sum(tp_ref[:], axis=0)
    dis = dis_ref[:]
    s = dis * (t + dis * st_ref[:])
    inv_n = jnp.float32(1.0 / n)
    m = jnp.sum(s) * inv_n
    var = jnp.sum(s * s) * inv_n - m * m
    w = w_ref[:]
    invstd = lax.rsqrt(var * w * w + 1e-5)
    cg = w * invstd * g_ref[:]
    a0c = jnp.sum(cg * lwt_ref[0:1, :])
    a1c = jnp.sum(cg * lwt_ref[1:2, :])
    b0c = jnp.sum(be_ref[:] * lwt_ref[0:1, :]) + lb_ref[0, 0]
    b1c = jnp.sum(be_ref[:] * lwt_ref[1:2, :]) + lb_ref[0, 1]
    z = s - m
    r0 = jnp.maximum(z * a0c + b0c, 0.0)
    r1 = jnp.maximum(z * a1c + b1c, 0.0)
    mx = jnp.maximum(r0, r1)
    e0 = jnp.exp(r0 - mx)
    e1 = jnp.exp(r1 - mx)
    tot = e0 + e1
    a0_ref[:] = e0 / tot
    a1_ref[:] = e1 / tot


def kernel(state, edge_index, edge_attr, W, b, gamma, beta, lin_W, lin_b):
    del edge_attr, b
    n = state.shape[0]
    e = edge_index.shape[1]
    hidden = W.shape[1]
    np_ = _round_up(n, 2048)
    rows = np_ // 128

    state_p = jnp.concatenate([state, jnp.zeros((np_ - n,), state.dtype)])
    state_2d = state_p.reshape(rows, 128)

    edges_flat = edge_index.reshape(2 * e)
    deg_parts = _make_deg_kernel(np_, e)(edges_flat)

    u2d, dis2d = pl.pallas_call(
        _u_body,
        out_shape=[
            jax.ShapeDtypeStruct((rows, 128), jnp.float32),
            jax.ShapeDtypeStruct((rows, 128), jnp.float32),
        ],
    )(deg_parts.reshape(_NWORKERS, rows, 128), state_2d)

    t_parts = _make_agg_kernel(np_, e)(edges_flat, u2d.reshape(np_))

    a0, a1 = pl.pallas_call(
        functools.partial(_head_body, n),
        out_shape=[
            jax.ShapeDtypeStruct((rows, 128), jnp.float32),
            jax.ShapeDtypeStruct((rows, 128), jnp.float32),
        ],
    )(
        t_parts.reshape(_NWORKERS, rows, 128),
        dis2d,
        state_2d,
        W.reshape(1, hidden),
        gamma.reshape(1, hidden),
        beta.reshape(1, hidden),
        lin_W.T.reshape(2, hidden),
        lin_b.reshape(1, 2),
    )
    return jnp.stack([a0.reshape(-1)[:n], a1.reshape(-1)[:n]], axis=1)


# speedup vs baseline: 195.6168x; 1.1853x over previous
import functools

import jax
import jax.numpy as jnp
from jax import lax
from jax.experimental import pallas as pl
from jax.experimental.pallas import tpu as pltpu
from jax.experimental.pallas import tpu_sc as plsc

_LANES = 16
_CH = 3200
_NWORKERS = 32
_UNROLL = 4


def _round_up(x, m):
    return (x + m - 1) // m * m


def _zero_ref(ref, n):
    zeros = jnp.zeros((_LANES,), jnp.float32)
    groups = n // _LANES
    uz = 8

    def body(i, c):
        for k in range(uz):
            ref[pl.ds((i * uz + k) * _LANES, _LANES)] = zeros
        return c

    lax.fori_loop(0, groups // uz, body, None)
    for g in range(groups - groups % uz, groups):
        ref[pl.ds(g * _LANES, _LANES)] = zeros


def _emit_groups(ngroups, group_fn):
    main = ngroups - ngroups % _UNROLL

    def body(i, c):
        for k in range(_UNROLL):
            group_fn(i * _UNROLL + k)
        return c

    if main:
        lax.fori_loop(0, main // _UNROLL, body, None)
    for g in range(main, ngroups):
        group_fn(g)


def _make_deg_kernel(np_, e):
    epw = e // _NWORKERS
    nch = epw // _CH
    rem = epw % _CH
    rem_groups = rem // _LANES
    rem_tail = rem % _LANES
    mesh = plsc.VectorSubcoreMesh(core_axis_name="c", subcore_axis_name="s")

    nchunks = nch + (1 if rem else 0)

    def _size(ci):
        return _CH if ci < nch else rem

    @functools.partial(
        pl.kernel,
        out_type=jax.ShapeDtypeStruct((_NWORKERS, np_), jnp.float32),
        mesh=mesh,
        scratch_types=[
            pltpu.VMEM((np_,), jnp.float32),
            pltpu.VMEM((2 * _CH,), jnp.int32),
            pltpu.SemaphoreType.DMA((2,)),
        ],
        compiler_params=pltpu.CompilerParams(needs_layout_passes=False),
    )
    def deg_kernel(edge_hbm, out_hbm, acc, dbuf, dsem):
        wid = lax.axis_index("c") * 16 + lax.axis_index("s")
        ones = jnp.ones((_LANES,), jnp.float32)
        base = e + wid * epw

        def _pair(ci):
            sz = _size(ci)
            b = ci % 2
            return (edge_hbm.at[pl.ds(base + ci * _CH, sz)],
                    dbuf.at[pl.ds(b * _CH, sz)], dsem.at[b])

        pltpu.async_copy(*_pair(0))
        _zero_ref(acc, np_)
        for ci in range(nchunks):
            if ci + 1 < nchunks:
                pltpu.async_copy(*_pair(ci + 1))
            pltpu.make_async_copy(*_pair(ci)).wait()
            b = ci % 2

            def scat(g, b=b):
                idx = dbuf[pl.ds(b * _CH + g * _LANES, _LANES)]
                plsc.addupdate_scatter(acc, [idx], ones)

            _emit_groups(_size(ci) // _LANES, scat)
        if rem_tail:
            m = lax.iota(jnp.int32, 16) < rem_tail
            idx = dbuf[pl.ds(((nchunks - 1) % 2) * _CH + rem_groups * _LANES,
                             _LANES)]
            plsc.addupdate_scatter(acc, [idx], ones, mask=m)
        pltpu.sync_copy(acc, out_hbm.at[wid])

    return deg_kernel


def _make_agg_kernel(np_, e):
    epw = e // _NWORKERS
    nch = epw // _CH
    rem = epw % _CH
    rem_groups = rem // _LANES
    rem_tail = rem % _LANES
    mesh = plsc.VectorSubcoreMesh(core_axis_name="c", subcore_axis_name="s")

    nchunks = nch + (1 if rem else 0)

    def _size(ci):
        return _CH if ci < nch else rem

    @functools.partial(
        pl.kernel,
        out_type=jax.ShapeDtypeStruct((_NWORKERS, np_), jnp.float32),
        mesh=mesh,
        scratch_types=[
            pltpu.VMEM((np_,), jnp.float32),
            pltpu.VMEM((np_,), jnp.float32),
            pltpu.VMEM((2 * _CH,), jnp.int32),
            pltpu.VMEM((2 * _CH,), jnp.int32),
            pltpu.SemaphoreType.DMA((2,)),
            pltpu.SemaphoreType.DMA((2,)),
            pltpu.SemaphoreType.DMA,
        ],
        compiler_params=pltpu.CompilerParams(needs_layout_passes=False),
    )
    def agg_kernel(edge_hbm, u_hbm, out_hbm, u, tacc, sbuf, dbuf,
                   ssem, dsem, usem):
        wid = lax.axis_index("c") * 16 + lax.axis_index("s")
        base = wid * epw

        def _spair(ci):
            sz = _size(ci)
            b = ci % 2
            return (edge_hbm.at[pl.ds(base + ci * _CH, sz)],
                    sbuf.at[pl.ds(b * _CH, sz)], ssem.at[b])

        def _dpair(ci):
            sz = _size(ci)
            b = ci % 2
            return (edge_hbm.at[pl.ds(e + base + ci * _CH, sz)],
                    dbuf.at[pl.ds(b * _CH, sz)], dsem.at[b])

        pltpu.async_copy(*_spair(0))
        pltpu.async_copy(*_dpair(0))
        pltpu.async_copy(u_hbm, u, usem)
        _zero_ref(tacc, np_)
        pltpu.make_async_copy(u_hbm, u, usem).wait()
        for ci in range(nchunks):
            if ci + 1 < nchunks:
                pltpu.async_copy(*_spair(ci + 1))
                pltpu.async_copy(*_dpair(ci + 1))
            pltpu.make_async_copy(*_spair(ci)).wait()
            pltpu.make_async_copy(*_dpair(ci)).wait()
            b = ci % 2

            def gs(g, b=b):
                off = b * _CH + g * _LANES
                vals = plsc.load_gather(u, [sbuf[pl.ds(off, _LANES)]])
                plsc.addupdate_scatter(tacc, [dbuf[pl.ds(off, _LANES)]], vals)

            _emit_groups(_size(ci) // _LANES, gs)
        if rem_tail:
            m = lax.iota(jnp.int32, 16) < rem_tail
            off = ((nchunks - 1) % 2) * _CH + rem_groups * _LANES
            ds = pl.ds(off, _LANES)
            vals = plsc.load_gather(u, [sbuf[ds]], mask=m)
            plsc.addupdate_scatter(tacc, [dbuf[ds]], vals, mask=m)
        pltpu.sync_copy(tacc, out_hbm.at[wid])

    return agg_kernel


def _u_body(degp_ref, state_ref, u_ref, dis_ref):
    deg = jnp.sum(degp_ref[:], axis=0) + 1.0
    dis = lax.rsqrt(deg)
    dis_ref[:] = dis
    u_ref[:] = dis * state_ref[:]


def _head_body(n, tp_ref, dis_ref, st_ref, w_ref, g_ref, be_ref, lwt_ref,
               lb_ref, a0_ref, a1_ref):
    t = jnp.sum(tp_ref[:], axis=0)
    dis = dis_ref[:]
    s = dis * (t + dis * st_ref[:])
    inv_n = jnp.float32(1.0 / n)
    m = jnp.sum(s) * inv_n
    var = jnp.sum(s * s) * inv_n - m * m
    w = w_ref[:]
    invstd = lax.rsqrt(var * w * w + 1e-5)
    cg = w * invstd * g_ref[:]
    a0c = jnp.sum(cg * lwt_ref[0:1, :])
    a1c = jnp.sum(cg * lwt_ref[1:2, :])
    b0c = jnp.sum(be_ref[:] * lwt_ref[0:1, :]) + lb_ref[0, 0]
    b1c = jnp.sum(be_ref[:] * lwt_ref[1:2, :]) + lb_ref[0, 1]
    z = s - m
    r0 = jnp.maximum(z * a0c + b0c, 0.0)
    r1 = jnp.maximum(z * a1c + b1c, 0.0)
    mx = jnp.maximum(r0, r1)
    e0 = jnp.exp(r0 - mx)
    e1 = jnp.exp(r1 - mx)
    tot = e0 + e1
    a0_ref[:] = e0 / tot
    a1_ref[:] = e1 / tot


def kernel(state, edge_index, edge_attr, W, b, gamma, beta, lin_W, lin_b):
    del edge_attr, b
    n = state.shape[0]
    e = edge_index.shape[1]
    hidden = W.shape[1]
    np_ = _round_up(n, 2048)
    rows = np_ // 128

    state_p = jnp.concatenate([state, jnp.zeros((np_ - n,), state.dtype)])
    state_2d = state_p.reshape(rows, 128)

    edges_flat = edge_index.reshape(2 * e)
    deg_parts = _make_deg_kernel(np_, e)(edges_flat)

    u2d, dis2d = pl.pallas_call(
        _u_body,
        out_shape=[
            jax.ShapeDtypeStruct((rows, 128), jnp.float32),
            jax.ShapeDtypeStruct((rows, 128), jnp.float32),
        ],
    )(deg_parts.reshape(_NWORKERS, rows, 128), state_2d)

    t_parts = _make_agg_kernel(np_, e)(edges_flat, u2d.reshape(np_))

    a0, a1 = pl.pallas_call(
        functools.partial(_head_body, n),
        out_shape=[
            jax.ShapeDtypeStruct((rows, 128), jnp.float32),
            jax.ShapeDtypeStruct((rows, 128), jnp.float32),
        ],
    )(
        t_parts.reshape(_NWORKERS, rows, 128),
        dis2d,
        state_2d,
        W.reshape(1, hidden),
        gamma.reshape(1, hidden),
        beta.reshape(1, hidden),
        lin_W.T.reshape(2, hidden),
        lin_b.reshape(1, 2),
    )
    return jnp.stack([a0.reshape(-1)[:n], a1.reshape(-1)[:n]], axis=1)
